# baseline (speedup 1.0000x reference)
import functools

import jax
import jax.numpy as jnp
from jax import lax
from jax.experimental import pallas as pl
from jax.experimental.pallas import tpu as pltpu
from jax.experimental.pallas import tpu_sc as plsc

K_TOP = 64
SCALE = 2.0
M_TILE = 512
N_SEARCH = 16
RANK = 512


def _stage_a_body(x_ref, at_ref, z_ref):
    z_ref[...] = jnp.dot(x_ref[...], at_ref[...], preferred_element_type=jnp.float32)


def _stage_c_body(x_ref, z_ref, th_ref, wt_ref, bwt_ref, b_ref, out_ref):
    t = th_ref[0]
    zm = jnp.where(z_ref[...] >= t, z_ref[...], 0.0)
    out = jnp.dot(x_ref[...], wt_ref[...], preferred_element_type=jnp.float32)
    out = out + b_ref[...]
    out = out + SCALE * jnp.dot(zm, bwt_ref[...], preferred_element_type=jnp.float32)
    out_ref[...] = out


def _sc_thresholds(z):
    n = z.shape[0]
    info = plsc.get_sparse_core_info()
    nc, ns, lanes = info.num_cores, info.num_subcores, info.num_lanes
    nw = nc * ns
    per_w = n // nw
    chunk = 128
    n_chunks = per_w // chunk
    nv = RANK // lanes

    mesh = plsc.VectorSubcoreMesh(core_axis_name="c", subcore_axis_name="s")

    @functools.partial(
        pl.kernel,
        mesh=mesh,
        out_type=jax.ShapeDtypeStruct((n,), jnp.float32),
        scratch_types=[
            pltpu.VMEM((chunk, RANK), jnp.float32),
            pltpu.VMEM((chunk,), jnp.float32),
        ],
    )
    def sc_kernel(z_hbm, th_hbm, zc, tc):
        wid = lax.axis_index("s") * nc + lax.axis_index("c")
        base = wid * per_w

        def chunk_body(ci, carry):
            cbase = base + ci * chunk
            pltpu.sync_copy(z_hbm.at[pl.ds(cbase, chunk)], zc)

            lane = lax.iota(jnp.int32, 16)
            perms = [lane ^ s for s in (8, 4, 2, 1)]

            def allreduce(v, op):
                for p in perms:
                    v = op(v, v.at[p].get(mode="promise_in_bounds"))
                return v

            def group_body(g, carry2):
                def tok_body(j, th_acc):
                    t = g * lanes + j
                    vals = [zc[t, pl.ds(i * lanes, lanes)] for i in range(nv)]
                    mn = vals[0]
                    mx = vals[0]
                    for v in vals[1:]:
                        mn = jnp.minimum(mn, v)
                        mx = jnp.maximum(mx, v)
                    lo = allreduce(mn, jnp.minimum)
                    hi = allreduce(mx, jnp.maximum)
                    for _ in range(N_SEARCH):
                        mid = 0.5 * (lo + hi)
                        acc = None
                        for v in vals:
                            one = jnp.where(v >= mid, 1.0, 0.0)
                            acc = one if acc is None else acc + one
                        cnt = allreduce(acc, jnp.add)
                        pred = cnt >= float(K_TOP)
                        lo = jnp.where(pred, mid, lo)
                        hi = jnp.where(pred, hi, mid)
                    return jnp.where(lane == j, lo, th_acc)

                th_vec = lax.fori_loop(0, lanes, tok_body, jnp.zeros((lanes,), jnp.float32))
                tc[pl.ds(g * lanes, lanes)] = th_vec
                return carry2

            lax.fori_loop(0, chunk // lanes, group_body, 0)
            pltpu.sync_copy(tc, th_hbm.at[pl.ds(cbase, chunk)])
            return carry

        lax.fori_loop(0, n_chunks, chunk_body, 0)

    return sc_kernel(z)


def kernel(x, A, Bw, W, b):
    batch, seq, d_in = x.shape
    n = batch * seq
    r = A.shape[0]
    d_out = W.shape[0]
    x2 = x.reshape(n, d_in)

    z = pl.pallas_call(
        _stage_a_body,
        grid=(n // M_TILE,),
        in_specs=[
            pl.BlockSpec((M_TILE, d_in), lambda i: (i, 0)),
            pl.BlockSpec((d_in, r), lambda i: (0, 0)),
        ],
        out_specs=pl.BlockSpec((M_TILE, r), lambda i: (i, 0)),
        out_shape=jax.ShapeDtypeStruct((n, r), jnp.float32),
        compiler_params=pltpu.CompilerParams(
            dimension_semantics=("parallel",),
        ),
    )(x2, A.T)

    th = _sc_thresholds(z).reshape(n // M_TILE, M_TILE, 1)

    out = pl.pallas_call(
        _stage_c_body,
        grid=(n // M_TILE,),
        in_specs=[
            pl.BlockSpec((M_TILE, d_in), lambda i: (i, 0)),
            pl.BlockSpec((M_TILE, r), lambda i: (i, 0)),
            pl.BlockSpec((1, M_TILE, 1), lambda i: (i, 0, 0)),
            pl.BlockSpec((d_in, d_out), lambda i: (0, 0)),
            pl.BlockSpec((r, d_out), lambda i: (0, 0)),
            pl.BlockSpec((1, d_out), lambda i: (0, 0)),
        ],
        out_specs=pl.BlockSpec((M_TILE, d_out), lambda i: (i, 0)),
        out_shape=jax.ShapeDtypeStruct((n, d_out), jnp.float32),
        compiler_params=pltpu.CompilerParams(
            dimension_semantics=("parallel",),
        ),
    )(x2, z, th, W.T, Bw.T, b.reshape(1, d_out))
    return out.reshape(batch, seq, d_out)


# speedup vs baseline: 4.5151x; 4.5151x over previous
import jax
import jax.numpy as jnp
from jax.experimental import pallas as pl
from jax.experimental.pallas import tpu as pltpu

K_TOP = 64
SCALE = 2.0
M_TILE = 1024
N_SEARCH = 16


def _fused_body(x_ref, a_ref, w_ref, bw_ref, b_ref, out_ref):
    x = x_ref[...]
    xt = x.T
    zt = jnp.dot(a_ref[...], xt, preferred_element_type=jnp.float32)

    ot = jnp.dot(w_ref[...], xt, preferred_element_type=jnp.float32)
    ot = ot + b_ref[...]

    lo = jnp.min(zt, axis=0, keepdims=True)
    hi = jnp.max(zt, axis=0, keepdims=True)

    def body(_, carry):
        lo, hi = carry
        mid = 0.5 * (lo + hi)
        cnt = jnp.sum((zt >= mid).astype(jnp.float32), axis=0, keepdims=True)
        pred = cnt >= float(K_TOP)
        return jnp.where(pred, mid, lo), jnp.where(pred, hi, mid)

    lo, hi = jax.lax.fori_loop(0, N_SEARCH, body, (lo, hi))

    zmt = jnp.where(zt >= lo, zt, 0.0)
    ot = ot + SCALE * jnp.dot(bw_ref[...], zmt, preferred_element_type=jnp.float32)
    out_ref[...] = ot.T


def kernel(x, A, Bw, W, b):
    batch, seq, d_in = x.shape
    n = batch * seq
    r = A.shape[0]
    d_out = W.shape[0]
    x2 = x.reshape(n, d_in)

    out = pl.pallas_call(
        _fused_body,
        grid=(n // M_TILE,),
        in_specs=[
            pl.BlockSpec((M_TILE, d_in), lambda i: (i, 0)),
            pl.BlockSpec((r, d_in), lambda i: (0, 0)),
            pl.BlockSpec((d_out, d_in), lambda i: (0, 0)),
            pl.BlockSpec((d_out, r), lambda i: (0, 0)),
            pl.BlockSpec((d_out, 1), lambda i: (0, 0)),
        ],
        out_specs=pl.BlockSpec((M_TILE, d_out), lambda i: (i, 0)),
        out_shape=jax.ShapeDtypeStruct((n, d_out), jnp.float32),
        compiler_params=pltpu.CompilerParams(
            dimension_semantics=("parallel",),
        ),
    )(x2, A, W, Bw, b.reshape(d_out, 1))
    return out.reshape(batch, seq, d_out)
